# baseline (speedup 1.0000x reference)
import functools

import jax
import jax.numpy as jnp
from jax import lax
from jax.experimental import pallas as pl
from jax.experimental.pallas import tpu as pltpu
from jax.experimental.pallas import tpu_sc as plsc

NUM_FRAMES = 16
T = 512
FRAME_ELEMS = 3 * 224 * 224
HALVES = 2
ROW = FRAME_ELEMS // HALVES
NC, NS = 2, 16
NW = NC * NS


def _sc_gather(vid2, ridx):
    mesh = plsc.VectorSubcoreMesh(core_axis_name="c", subcore_axis_name="s")

    @functools.partial(
        pl.kernel,
        out_type=jax.ShapeDtypeStruct((NW, ROW), jnp.float32),
        mesh=mesh,
        scratch_types=[
            pltpu.VMEM((8,), jnp.int32),
            pltpu.VMEM((1, ROW), jnp.float32),
            pltpu.SemaphoreType.DMA,
        ],
    )
    def k(vid_hbm, ridx_hbm, out_hbm, idx_v, buf_v, sem):
        wid = lax.axis_index("s") * NC + lax.axis_index("c")
        pltpu.sync_copy(ridx_hbm.at[pl.ds(wid * 8, 8)], idx_v)
        pltpu.async_copy(vid_hbm.at[idx_v.at[pl.ds(0, 1)]], buf_v, sem).wait()
        pltpu.sync_copy(buf_v, out_hbm.at[pl.ds(wid, 1)])

    return k(vid2, ridx)


def kernel(vid):
    tix = jax.random.randint(jax.random.key(42), (NUM_FRAMES,), 0, vid.shape[0])
    rows = (tix.astype(jnp.int32)[:, None] * HALVES
            + jnp.arange(HALVES, dtype=jnp.int32)[None, :]).reshape(NW)
    ridx = jnp.zeros((NW * 8,), jnp.int32).at[jnp.arange(NW) * 8].set(rows)
    vid2 = vid.reshape(T * HALVES, ROW)
    out = _sc_gather(vid2, ridx)
    return out.reshape(NUM_FRAMES, 3, 224, 224)


# speedup vs baseline: 2.1739x; 2.1739x over previous
import functools

import jax
import jax.numpy as jnp
from jax import lax
from jax.experimental import pallas as pl
from jax.experimental.pallas import tpu as pltpu
from jax.experimental.pallas import tpu_sc as plsc

NUM_FRAMES = 16
T = 512
C = 3
H = W = 224
HH = H // 2
NC, NS = 2, 16
NW = NC * NS
NQ = NUM_FRAMES * C * 2


def _sc_gather(table, src_idx):
    mesh = plsc.VectorSubcoreMesh(core_axis_name="c", subcore_axis_name="s")

    @functools.partial(
        pl.kernel,
        out_type=jax.ShapeDtypeStruct((NUM_FRAMES * C, H, W), jnp.float32),
        mesh=mesh,
        scratch_types=[
            pltpu.VMEM((1, 128), jnp.int32),
            pltpu.VMEM((1, HH, W), jnp.float32),
            pltpu.VMEM((1, HH, W), jnp.float32),
            pltpu.VMEM((1, HH, W), jnp.float32),
            pltpu.SemaphoreType.DMA,
            pltpu.SemaphoreType.DMA,
            pltpu.SemaphoreType.DMA,
            pltpu.SemaphoreType.DMA,
            pltpu.SemaphoreType.DMA,
            pltpu.SemaphoreType.DMA,
        ],
        compiler_params=pltpu.CompilerParams(
            use_tc_tiling_on_sc=True, needs_layout_passes=False),
    )
    def k(tab_hbm, idx_hbm, out_hbm, idx_v, b0, b1, b2,
          g0, g1, g2, w0, w1, w2):
        wid = lax.axis_index("s") * NC + lax.axis_index("c")
        bufs, gsem, wsem = (b0, b1, b2), (g0, g1, g2), (w0, w1, w2)
        pltpu.sync_copy(idx_hbm.at[pl.ds(wid, 1)], idx_v)
        vec = idx_v[0, pl.ds(0, 16)]
        lanes = lax.iota(jnp.int32, 16)
        gathers = []
        for j in range(3):
            s = jnp.max(jnp.where(lanes == j, vec, 0))
            h = ((wid + j) % 2) * HH
            cp = pltpu.make_async_copy(
                tab_hbm.at[pl.ds(s, 1), pl.ds(h, HH)], bufs[j], gsem[j])
            cp.start()
            gathers.append(cp)
        writes = []
        for j in range(3):
            gathers[j].wait()
            q = 3 * wid + j
            cp = pltpu.make_async_copy(
                bufs[j],
                out_hbm.at[pl.ds(q // 2, 1), pl.ds((q % 2) * HH, HH)],
                wsem[j])
            cp.start()
            writes.append(cp)
        for cp in writes:
            cp.wait()

    return k(table, src_idx)


def kernel(vid):
    tix = jax.random.randint(jax.random.key(42), (NUM_FRAMES,), 0, vid.shape[0])
    q = jnp.arange(NQ, dtype=jnp.int32)
    src = tix.astype(jnp.int32)[q // (2 * C)] * C + (q % (2 * C)) // 2
    src_idx = jnp.zeros((NW, 128), jnp.int32).at[:, :3].set(src.reshape(NW, 3))
    table = vid.reshape(T * C, H, W)
    out = _sc_gather(table, src_idx)
    return out.reshape(NUM_FRAMES, C, H, W)
